# baseline (speedup 1.0000x reference)
import dataclasses
import functools

import jax
import jax.numpy as jnp
from jax import lax
from jax.experimental import pallas as pl
from jax.experimental.pallas import tpu as pltpu
from jax.experimental.pallas import tpu_sc as plsc

N = 10000
E = 320000
D = 128
NP = 10240
NTILES = 32
NB = NP // NTILES
SC_CH = 1280
CAPL = 11264
KB = 128
BN = 1280
PK = 16384
BE = 2560


def _prep_body(x_ref, wfc_ref, wattn_ref, z_ref, as_ref, ad_ref):
    xb = x_ref[...]
    w = wfc_ref[...]
    zb = lax.dot_general(xb, w, (((1,), (1,)), ((), ())),
                         preferred_element_type=jnp.float32)
    z_ref[...] = zb
    wa = wattn_ref[...]
    w1 = wa[0, :D]
    w2 = wa[0, D:]
    a_s = lax.dot_general(zb, w1, (((1,), (0,)), ((), ())),
                          preferred_element_type=jnp.float32)
    a_d = lax.dot_general(zb, w2, (((1,), (0,)), ((), ())),
                          preferred_element_type=jnp.float32)
    pad = jnp.zeros((7, a_s.shape[0]), jnp.float32)
    as_ref[...] = jnp.concatenate([a_s[None], pad], axis=0)
    ad_ref[...] = jnp.concatenate([a_d[None], pad], axis=0)


def _tc_prep(xp, W_fc, W_attn):
    return pl.pallas_call(
        _prep_body,
        grid=(NP // BN,),
        in_specs=[
            pl.BlockSpec((BN, D), lambda i: (i, 0)),
            pl.BlockSpec((D, D), lambda i: (0, 0)),
            pl.BlockSpec((1, 2 * D), lambda i: (0, 0)),
        ],
        out_specs=[
            pl.BlockSpec((BN, D), lambda i: (i, 0)),
            pl.BlockSpec((8, BN), lambda i: (0, i)),
            pl.BlockSpec((8, BN), lambda i: (0, i)),
        ],
        out_shape=[
            jax.ShapeDtypeStruct((NP, D), jnp.float32),
            jax.ShapeDtypeStruct((8, NP), jnp.float32),
            jax.ShapeDtypeStruct((8, NP), jnp.float32),
        ],
    )(xp, W_fc, W_attn)


def _pack_body(ei_ref, p_ref):
    src = ei_ref[0, :]
    dst = ei_ref[1, :]
    p = dst * PK + src
    p_ref[...] = jnp.broadcast_to(p[None], (8, p.shape[0]))


def _tc_pack(edge_index):
    return pl.pallas_call(
        _pack_body,
        grid=(E // BE,),
        in_specs=[pl.BlockSpec((2, BE), lambda i: (0, i))],
        out_specs=pl.BlockSpec((8, BE), lambda i: (0, i)),
        out_shape=jax.ShapeDtypeStruct((8, E), jnp.int32),
    )(edge_index)


def _gat_sc_body(z_hbm, as_hbm, ad_hbm, p_hbm, out_hbm,
                 as_v, ad_v, den_v, qL, relL, wL, h_loc, zb0, zb1,
                 pb0, pb1, abuf,
                 sem_s0, sem_s1, sem_z0, sem_z1):
    wid = lax.axis_index("s") * 2 + lax.axis_index("c")
    base = wid * NB
    lo = base * PK
    hi = (base + NB) * PK
    cp_as = pltpu.make_async_copy(as_hbm.at[0], as_v, sem_s0)
    cp_ad = pltpu.make_async_copy(ad_hbm.at[0], ad_v, sem_s1)
    cp_as.start()
    cp_ad.start()

    zeros16 = jnp.zeros((16,), jnp.float32)
    izeros16 = jnp.zeros((16,), jnp.int32)

    @pl.loop(0, NB)
    def _(r):
        for g in range(D // 16):
            h_loc[r, pl.ds(g * 16, 16)] = zeros16

    @pl.loop(0, NB, step=16)
    def _(i):
        den_v[pl.ds(i, 16)] = zeros16

    cp_as.wait()
    cp_ad.wait()

    def _mx(i, m):
        return jnp.maximum(m, as_v[pl.ds(i * 16, 16)])
    mvec = lax.fori_loop(0, NP // 16, _mx,
                         jnp.full((16,), -1e30, jnp.float32))
    A = jnp.max(mvec)

    lane_iota = lax.iota(jnp.int32, 16)

    NCH = E // SC_CH

    def _scan_cp(c, pb, sem):
        return pltpu.make_async_copy(
            p_hbm.at[pl.ds(c * SC_CH, SC_CH)], pb, sem)

    _scan_cp(0, pb0, sem_s0).start()
    _scan_cp(1, pb1, sem_s1).start()

    def _scan_vecs(pb, off):
        @plsc.parallel_loop(0, SC_CH // 16, unroll=2, carry=off)
        def vec_body(v, off):
            pv = pb[pl.ds(v * 16, 16)]
            msk = (pv >= lo) & (pv < hi)
            cnt = plsc.all_reduce_population_count(msk)[0]
            off_c = jnp.minimum(off, CAPL - 16)
            srcv = pv & (PK - 1)
            relv = (pv >> 14) - base
            plsc.store_compressed(qL.at[pl.ds(off_c, 16)], srcv, mask=msk)
            plsc.store_compressed(relL.at[pl.ds(off_c, 16)], relv, mask=msk)
            return off + cnt
        return vec_body

    def scan_pair(c2, off):
        c = c2 * 2
        _scan_cp(c, pb0, sem_s0).wait()
        off = _scan_vecs(pb0, off)

        @pl.when(c + 2 < NCH)
        def _():
            _scan_cp(c + 2, pb0, sem_s0).start()

        _scan_cp(c + 1, pb1, sem_s1).wait()
        off = _scan_vecs(pb1, off)

        @pl.when(c + 3 < NCH)
        def _():
            _scan_cp(c + 3, pb1, sem_s1).start()
        return off

    M = lax.fori_loop(0, NCH // 2, scan_pair, jnp.int32(0))
    M = jnp.minimum(M, CAPL - 16)

    i0 = (M // 16) * 16
    keep = lane_iota < (M - i0)
    qL[pl.ds(i0, 16)] = jnp.where(keep, qL[pl.ds(i0, 16)], 0)
    relL[pl.ds(i0, 16)] = jnp.where(keep, relL[pl.ds(i0, 16)], 0)

    def zero_tail(t, _):
        i = jnp.minimum(i0 + 16 + t * 16, CAPL - 16)
        qL[pl.ds(i, 16)] = izeros16
        relL[pl.ds(i, 16)] = izeros16
        return 0
    lax.fori_loop(0, KB // 16, zero_tail, 0)

    def den_body(i, _):
        srcv = qL[pl.ds(i * 16, 16)]
        relv = relL[pl.ds(i * 16, 16)]
        asv = plsc.load_gather(as_v, [srcv])
        adv = plsc.load_gather(ad_v, [relv + base])
        s = asv + adv
        e = jnp.maximum(s, 0.01 * s)
        m = A + adv
        m = jnp.maximum(m, 0.01 * m)
        w = jnp.exp(e - m)
        wL[pl.ds(i * 16, 16)] = w
        vm = (lane_iota + i * 16) < M
        plsc.addupdate_scatter(den_v, [relv], w, mask=vm)
        return 0
    lax.fori_loop(0, (M + 15) // 16, den_body, 0)

    nch = (M + KB - 1) // KB

    def _zcp(c, zb, sem):
        return pltpu.make_async_copy(z_hbm.at[qL.at[pl.ds(c * KB, KB)]],
                                     zb, sem)

    @pl.when(nch > 0)
    def _():
        _zcp(0, zb0, sem_z0).start()

    @pl.when(nch > 1)
    def _():
        _zcp(1, zb1, sem_z1).start()

    def _process(c, zb):
        def alpha_g(g, _):
            i = c * (KB // 16) + g
            relv = relL[pl.ds(i * 16, 16)]
            wv = wL[pl.ds(i * 16, 16)]
            den = plsc.load_gather(den_v, [relv]) + 1e-16
            a = wv / den
            vm = (lane_iota + i * 16) < M
            a = jnp.where(vm, a, 0.0)
            abuf[pl.ds(g * 16, 16)] = a
            return 0
        lax.fori_loop(0, KB // 16, alpha_g, 0, unroll=4)

        @plsc.parallel_loop(0, KB // 16)
        def group_body(g):
            av = abuf[pl.ds(g * 16, 16)]
            relv = relL[pl.ds(c * KB + g * 16, 16)]
            for j in range(16):
                a = av[j]
                rel = relv[j]
                zvs = [zb[g * 16 + j, pl.ds(seg * 16, 16)]
                       for seg in range(D // 16)]
                for seg in range(D // 16):
                    plsc.addupdate(h_loc.at[rel, pl.ds(seg * 16, 16)],
                                   a * zvs[seg])

    def chunk_pair(c2, _):
        c = c2 * 2
        _zcp(c, zb0, sem_z0).wait()
        _process(c, zb0)

        @pl.when(c + 2 < nch)
        def _():
            _zcp(c + 2, zb0, sem_z0).start()

        @pl.when(c + 1 < nch)
        def _():
            _zcp(c + 1, zb1, sem_z1).wait()
            _process(c + 1, zb1)

            @pl.when(c + 3 < nch)
            def _():
                _zcp(c + 3, zb1, sem_z1).start()
        return 0

    lax.fori_loop(0, (nch + 1) // 2, chunk_pair, 0)

    pltpu.sync_copy(h_loc, out_hbm.at[pl.ds(base, NB)])


_sc_mesh = plsc.VectorSubcoreMesh(core_axis_name="c", subcore_axis_name="s")

_sc_params = pltpu.CompilerParams()
if "needs_layout_passes" in pltpu.CompilerParams.__dataclass_fields__:
    _sc_params = dataclasses.replace(_sc_params, needs_layout_passes=False)

_gat_sc = functools.partial(
    pl.kernel,
    out_type=jax.ShapeDtypeStruct((NP, D), jnp.float32),
    mesh=_sc_mesh,
    compiler_params=_sc_params,
    scratch_types=[
        pltpu.VMEM((NP,), jnp.float32),
        pltpu.VMEM((NP,), jnp.float32),
        pltpu.VMEM((NB,), jnp.float32),
        pltpu.VMEM((CAPL,), jnp.int32),
        pltpu.VMEM((CAPL,), jnp.int32),
        pltpu.VMEM((CAPL,), jnp.float32),
        pltpu.VMEM((NB, D), jnp.float32),
        pltpu.VMEM((KB, D), jnp.float32),
        pltpu.VMEM((KB, D), jnp.float32),
        pltpu.VMEM((SC_CH,), jnp.int32),
        pltpu.VMEM((SC_CH,), jnp.int32),
        pltpu.VMEM((KB,), jnp.float32),
        pltpu.SemaphoreType.DMA,
        pltpu.SemaphoreType.DMA,
        pltpu.SemaphoreType.DMA,
        pltpu.SemaphoreType.DMA,
    ],
)(_gat_sc_body)


def kernel(x, edge_index, W_fc, W_attn):
    xp = jnp.pad(x, ((0, NP - N), (0, 0)))
    z, as2, ad2 = _tc_prep(xp, W_fc, W_attn)
    p = _tc_pack(edge_index)[0]
    h = _gat_sc(z, as2, ad2, p)
    return h[:N]


# speedup vs baseline: 1.1678x; 1.1678x over previous
import dataclasses
import functools

import jax
import jax.numpy as jnp
from jax import lax
from jax.experimental import pallas as pl
from jax.experimental.pallas import tpu as pltpu
from jax.experimental.pallas import tpu_sc as plsc

N = 10000
E = 320000
D = 128
NP = 10240
NTILES = 32
NB = NP // NTILES
SC_CH = 1280
CAPL = 13312
KB = 128
BN = 1280


def _prep_body(x_ref, wfc_ref, wattn_ref, z_ref, as_ref, ad_ref):
    xb = x_ref[...]
    w = wfc_ref[...]
    zb = lax.dot_general(xb, w, (((1,), (1,)), ((), ())),
                         preferred_element_type=jnp.float32)
    z_ref[...] = zb
    wa = wattn_ref[...]
    w1 = wa[0, :D]
    w2 = wa[0, D:]
    a_s = lax.dot_general(zb, w1, (((1,), (0,)), ((), ())),
                          preferred_element_type=jnp.float32)
    a_d = lax.dot_general(zb, w2, (((1,), (0,)), ((), ())),
                          preferred_element_type=jnp.float32)
    pad = jnp.zeros((7, a_s.shape[0]), jnp.float32)
    as_ref[...] = jnp.concatenate([a_s[None], pad], axis=0)
    ad_ref[...] = jnp.concatenate([a_d[None], pad], axis=0)


def _tc_prep(xp, W_fc, W_attn):
    return pl.pallas_call(
        _prep_body,
        grid=(NP // BN,),
        in_specs=[
            pl.BlockSpec((BN, D), lambda i: (i, 0)),
            pl.BlockSpec((D, D), lambda i: (0, 0)),
            pl.BlockSpec((1, 2 * D), lambda i: (0, 0)),
        ],
        out_specs=[
            pl.BlockSpec((BN, D), lambda i: (i, 0)),
            pl.BlockSpec((8, BN), lambda i: (0, i)),
            pl.BlockSpec((8, BN), lambda i: (0, i)),
        ],
        out_shape=[
            jax.ShapeDtypeStruct((NP, D), jnp.float32),
            jax.ShapeDtypeStruct((8, NP), jnp.float32),
            jax.ShapeDtypeStruct((8, NP), jnp.float32),
        ],
    )(xp, W_fc, W_attn)


def _gat_sc_body(z_hbm, as_hbm, ad_hbm, src_hbm, dst_hbm, out_hbm,
                 as_v, ad_v, den_v, srcL, relL, h_loc, zb0, zb1,
                 sb0, db0, sb1, db1, abuf,
                 sem_s0, sem_d0, sem_s1, sem_d1, sem_z0, sem_z1):
    wid = lax.axis_index("s") * 2 + lax.axis_index("c")
    base = wid * NB
    cp_as = pltpu.make_async_copy(as_hbm.at[0], as_v, sem_s0)
    cp_ad = pltpu.make_async_copy(ad_hbm.at[0], ad_v, sem_d0)
    cp_as.start()
    cp_ad.start()

    zeros16 = jnp.zeros((16,), jnp.float32)
    izeros16 = jnp.zeros((16,), jnp.int32)

    @pl.loop(0, NB)
    def _(r):
        for g in range(D // 16):
            h_loc[r, pl.ds(g * 16, 16)] = zeros16

    @pl.loop(0, NB, step=16)
    def _(i):
        den_v[pl.ds(i, 16)] = zeros16

    cp_as.wait()
    cp_ad.wait()

    def _mx(i, m):
        return jnp.maximum(m, as_v[pl.ds(i * 16, 16)])
    mvec = lax.fori_loop(0, NP // 16, _mx,
                         jnp.full((16,), -1e30, jnp.float32))
    A = jnp.max(mvec)

    lane_iota = lax.iota(jnp.int32, 16)

    NCH = E // SC_CH

    def _scan_cp(c, sb, db, sem_s, sem_d):
        return (pltpu.make_async_copy(src_hbm.at[pl.ds(c * SC_CH, SC_CH)],
                                      sb, sem_s),
                pltpu.make_async_copy(dst_hbm.at[pl.ds(c * SC_CH, SC_CH)],
                                      db, sem_d))

    for cp in _scan_cp(0, sb0, db0, sem_s0, sem_d0):
        cp.start()
    for cp in _scan_cp(1, sb1, db1, sem_s1, sem_d1):
        cp.start()

    def _scan_vecs(sb, db, off):
        @plsc.parallel_loop(0, SC_CH // 16, unroll=2, carry=off)
        def vec_body(v, off):
            sv = sb[pl.ds(v * 16, 16)]
            dv = db[pl.ds(v * 16, 16)]
            rel = dv - base
            msk = rel.astype(jnp.uint32) < jnp.uint32(NB)
            cnt = plsc.all_reduce_population_count(msk)[0]
            off_c = jnp.minimum(off, CAPL - 16)
            plsc.store_compressed(srcL.at[pl.ds(off_c, 16)], sv, mask=msk)
            plsc.store_compressed(relL.at[pl.ds(off_c, 16)], rel, mask=msk)
            return off + cnt
        return vec_body

    def scan_pair(c2, off):
        c = c2 * 2
        for cp in _scan_cp(c, sb0, db0, sem_s0, sem_d0):
            cp.wait()
        off = _scan_vecs(sb0, db0, off)

        @pl.when(c + 2 < NCH)
        def _():
            for cp in _scan_cp(c + 2, sb0, db0, sem_s0, sem_d0):
                cp.start()

        for cp in _scan_cp(c + 1, sb1, db1, sem_s1, sem_d1):
            cp.wait()
        off = _scan_vecs(sb1, db1, off)

        @pl.when(c + 3 < NCH)
        def _():
            for cp in _scan_cp(c + 3, sb1, db1, sem_s1, sem_d1):
                cp.start()
        return off

    M = lax.fori_loop(0, NCH // 2, scan_pair, jnp.int32(0))
    M = jnp.minimum(M, CAPL - 16)

    i0 = (M // 16) * 16
    keep = lane_iota < (M - i0)
    srcL[pl.ds(i0, 16)] = jnp.where(keep, srcL[pl.ds(i0, 16)], 0)
    relL[pl.ds(i0, 16)] = jnp.where(keep, relL[pl.ds(i0, 16)], 0)

    def zero_tail(t, _):
        i = jnp.minimum(i0 + 16 + t * 16, CAPL - 16)
        srcL[pl.ds(i, 16)] = izeros16
        relL[pl.ds(i, 16)] = izeros16
        return 0
    lax.fori_loop(0, KB // 16, zero_tail, 0)

    def _edge_w(i):
        srcv = srcL[pl.ds(i * 16, 16)]
        relv = relL[pl.ds(i * 16, 16)]
        asv = plsc.load_gather(as_v, [srcv])
        adv = plsc.load_gather(ad_v, [relv + base])
        s = asv + adv
        e = jnp.maximum(s, 0.01 * s)
        m = A + adv
        m = jnp.maximum(m, 0.01 * m)
        w = jnp.exp(e - m)
        return relv, w

    def den_body(i, _):
        relv, w = _edge_w(i)
        vm = (lane_iota + i * 16) < M
        plsc.addupdate_scatter(den_v, [relv], w, mask=vm)
        return 0
    lax.fori_loop(0, (M + 15) // 16, den_body, 0)

    nch = (M + KB - 1) // KB

    def _zcp(c, zb, sem):
        return pltpu.make_async_copy(z_hbm.at[srcL.at[pl.ds(c * KB, KB)]],
                                     zb, sem)

    @pl.when(nch > 0)
    def _():
        _zcp(0, zb0, sem_z0).start()

    @pl.when(nch > 1)
    def _():
        _zcp(1, zb1, sem_z1).start()

    def _process(c, zb):
        def alpha_g(g, _):
            i = c * (KB // 16) + g
            relv, w = _edge_w(i)
            den = plsc.load_gather(den_v, [relv]) + 1e-16
            a = w / den
            vm = (lane_iota + i * 16) < M
            a = jnp.where(vm, a, 0.0)
            abuf[pl.ds(g * 16, 16)] = a
            return 0
        lax.fori_loop(0, KB // 16, alpha_g, 0, unroll=4)

        @plsc.parallel_loop(0, KB // 16)
        def group_body(g):
            av = abuf[pl.ds(g * 16, 16)]
            relv = relL[pl.ds(c * KB + g * 16, 16)]
            for j in range(16):
                a = av[j]
                rel = relv[j]
                zvs = [zb[g * 16 + j, pl.ds(seg * 16, 16)]
                       for seg in range(D // 16)]
                for seg in range(D // 16):
                    plsc.addupdate(h_loc.at[rel, pl.ds(seg * 16, 16)],
                                   a * zvs[seg])

    def chunk_pair(c2, _):
        c = c2 * 2
        _zcp(c, zb0, sem_z0).wait()
        _process(c, zb0)

        @pl.when(c + 2 < nch)
        def _():
            _zcp(c + 2, zb0, sem_z0).start()

        @pl.when(c + 1 < nch)
        def _():
            _zcp(c + 1, zb1, sem_z1).wait()
            _process(c + 1, zb1)

            @pl.when(c + 3 < nch)
            def _():
                _zcp(c + 3, zb1, sem_z1).start()
        return 0

    lax.fori_loop(0, (nch + 1) // 2, chunk_pair, 0)

    pltpu.sync_copy(h_loc, out_hbm.at[pl.ds(base, NB)])


_sc_mesh = plsc.VectorSubcoreMesh(core_axis_name="c", subcore_axis_name="s")

_sc_params = pltpu.CompilerParams()
if "needs_layout_passes" in pltpu.CompilerParams.__dataclass_fields__:
    _sc_params = dataclasses.replace(_sc_params, needs_layout_passes=False)

_gat_sc = functools.partial(
    pl.kernel,
    out_type=jax.ShapeDtypeStruct((NP, D), jnp.float32),
    mesh=_sc_mesh,
    compiler_params=_sc_params,
    scratch_types=[
        pltpu.VMEM((NP,), jnp.float32),
        pltpu.VMEM((NP,), jnp.float32),
        pltpu.VMEM((NB,), jnp.float32),
        pltpu.VMEM((CAPL,), jnp.int32),
        pltpu.VMEM((CAPL,), jnp.int32),
        pltpu.VMEM((NB, D), jnp.float32),
        pltpu.VMEM((KB, D), jnp.float32),
        pltpu.VMEM((KB, D), jnp.float32),
        pltpu.VMEM((SC_CH,), jnp.int32),
        pltpu.VMEM((SC_CH,), jnp.int32),
        pltpu.VMEM((SC_CH,), jnp.int32),
        pltpu.VMEM((SC_CH,), jnp.int32),
        pltpu.VMEM((KB,), jnp.float32),
        pltpu.SemaphoreType.DMA,
        pltpu.SemaphoreType.DMA,
        pltpu.SemaphoreType.DMA,
        pltpu.SemaphoreType.DMA,
        pltpu.SemaphoreType.DMA,
        pltpu.SemaphoreType.DMA,
    ],
)(_gat_sc_body)


def kernel(x, edge_index, W_fc, W_attn):
    xp = jnp.pad(x, ((0, NP - N), (0, 0)))
    z, as2, ad2 = _tc_prep(xp, W_fc, W_attn)
    h = _gat_sc(z, as2, ad2, edge_index[0], edge_index[1])
    return h[:N]


# speedup vs baseline: 1.1695x; 1.0015x over previous
import dataclasses
import functools

import jax
import jax.numpy as jnp
from jax import lax
from jax.experimental import pallas as pl
from jax.experimental.pallas import tpu as pltpu
from jax.experimental.pallas import tpu_sc as plsc

N = 10000
E = 320000
D = 128
NP = 10240
NTILES = 32
NB = NP // NTILES
SC_CH = 1280
CAPL = 13312
KB = 128
BN = 1280


def _prep_body(x_ref, wfc_ref, wattn_ref, z_ref, as_ref, ad_ref):
    xb = x_ref[...]
    w = wfc_ref[...]
    zb = lax.dot_general(xb, w, (((1,), (1,)), ((), ())),
                         preferred_element_type=jnp.float32)
    z_ref[...] = zb
    wa = wattn_ref[...]
    w1 = wa[0, :D]
    w2 = wa[0, D:]
    a_s = lax.dot_general(zb, w1, (((1,), (0,)), ((), ())),
                          preferred_element_type=jnp.float32)
    a_d = lax.dot_general(zb, w2, (((1,), (0,)), ((), ())),
                          preferred_element_type=jnp.float32)
    pad = jnp.zeros((7, a_s.shape[0]), jnp.float32)
    as_ref[...] = jnp.concatenate([a_s[None], pad], axis=0)
    ad_ref[...] = jnp.concatenate([a_d[None], pad], axis=0)


def _tc_prep(xp, W_fc, W_attn):
    return pl.pallas_call(
        _prep_body,
        grid=(NP // BN,),
        in_specs=[
            pl.BlockSpec((BN, D), lambda i: (i, 0)),
            pl.BlockSpec((D, D), lambda i: (0, 0)),
            pl.BlockSpec((1, 2 * D), lambda i: (0, 0)),
        ],
        out_specs=[
            pl.BlockSpec((BN, D), lambda i: (i, 0)),
            pl.BlockSpec((8, BN), lambda i: (0, i)),
            pl.BlockSpec((8, BN), lambda i: (0, i)),
        ],
        out_shape=[
            jax.ShapeDtypeStruct((NP, D), jnp.float32),
            jax.ShapeDtypeStruct((8, NP), jnp.float32),
            jax.ShapeDtypeStruct((8, NP), jnp.float32),
        ],
    )(xp, W_fc, W_attn)


def _gat_sc_body(z_hbm, as_hbm, ad_hbm, src_hbm, dst_hbm, out_hbm,
                 as_v, ad_v, den_v, srcL, relL, h_loc, zb0, zb1,
                 sb0, db0, sb1, db1, abuf,
                 sem_s0, sem_d0, sem_s1, sem_d1, sem_z0, sem_z1):
    wid = lax.axis_index("s") * 2 + lax.axis_index("c")
    base = wid * NB
    cp_as = pltpu.make_async_copy(as_hbm.at[0], as_v, sem_s0)
    cp_ad = pltpu.make_async_copy(ad_hbm.at[0], ad_v, sem_d0)
    cp_as.start()
    cp_ad.start()

    zeros16 = jnp.zeros((16,), jnp.float32)
    izeros16 = jnp.zeros((16,), jnp.int32)

    @pl.loop(0, NB)
    def _(r):
        for g in range(D // 16):
            h_loc[r, pl.ds(g * 16, 16)] = zeros16

    @pl.loop(0, NB, step=16)
    def _(i):
        den_v[pl.ds(i, 16)] = zeros16

    cp_as.wait()
    cp_ad.wait()

    def _mx(i, m):
        return jnp.maximum(m, as_v[pl.ds(i * 16, 16)])
    mvec = lax.fori_loop(0, NP // 16, _mx,
                         jnp.full((16,), -1e30, jnp.float32))
    A = jnp.max(mvec)

    lane_iota = lax.iota(jnp.int32, 16)

    NCH = E // SC_CH

    def _ecp(k, sb, db, sem_a, sem_b):
        return (pltpu.make_async_copy(src_hbm.at[pl.ds(k * SC_CH, SC_CH)],
                                      sb, sem_a),
                pltpu.make_async_copy(dst_hbm.at[pl.ds(k * SC_CH, SC_CH)],
                                      db, sem_b))

    for cp in _ecp(0, sb0, db0, sem_s0, sem_d0):
        cp.start()
    for cp in _ecp(1, sb1, db1, sem_s1, sem_d1):
        cp.start()

    def _scan_vecs(sb, db, off):
        @plsc.parallel_loop(0, SC_CH // 16, unroll=2, carry=off)
        def vec_body(v, off):
            sv = sb[pl.ds(v * 16, 16)]
            dv = db[pl.ds(v * 16, 16)]
            rel = dv - base
            msk = rel.astype(jnp.uint32) < jnp.uint32(NB)
            cnt = plsc.all_reduce_population_count(msk)[0]
            off_c = jnp.minimum(off, CAPL - 16)
            plsc.store_compressed(srcL.at[pl.ds(off_c, 16)], sv, mask=msk)
            plsc.store_compressed(relL.at[pl.ds(off_c, 16)], rel, mask=msk)
            return off + cnt
        return vec_body

    def chunk_pair_scan(k2, off):
        k = k2 * 2
        for cp in _ecp(k, sb0, db0, sem_s0, sem_d0):
            cp.wait()
        off = _scan_vecs(sb0, db0, off)

        @pl.when(k + 2 < NCH)
        def _():
            for cp in _ecp(k + 2, sb0, db0, sem_s0, sem_d0):
                cp.start()
        for cp in _ecp(k + 1, sb1, db1, sem_s1, sem_d1):
            cp.wait()
        off = _scan_vecs(sb1, db1, off)

        @pl.when(k + 3 < NCH)
        def _():
            for cp in _ecp(k + 3, sb1, db1, sem_s1, sem_d1):
                cp.start()
        return off

    M = lax.fori_loop(0, NCH // 2, chunk_pair_scan, jnp.int32(0))
    M = jnp.minimum(M, CAPL - 16)

    i0 = (M // 16) * 16
    keep = lane_iota < (M - i0)
    srcL[pl.ds(i0, 16)] = jnp.where(keep, srcL[pl.ds(i0, 16)], 0)
    relL[pl.ds(i0, 16)] = jnp.where(keep, relL[pl.ds(i0, 16)], 0)

    def zero_tail(t, _):
        i = jnp.minimum(i0 + 16 + t * 16, CAPL - 16)
        srcL[pl.ds(i, 16)] = izeros16
        relL[pl.ds(i, 16)] = izeros16
        return 0
    lax.fori_loop(0, KB // 16, zero_tail, 0)

    def _edge_w(i):
        srcv = srcL[pl.ds(i * 16, 16)]
        relv = relL[pl.ds(i * 16, 16)]
        asv = plsc.load_gather(as_v, [srcv])
        adv = plsc.load_gather(ad_v, [relv + base])
        s = asv + adv
        e = jnp.maximum(s, 0.01 * s)
        m = A + adv
        m = jnp.maximum(m, 0.01 * m)
        w = jnp.exp(e - m)
        return relv, w

    def den_body(i, _):
        relv, w = _edge_w(i)
        vm = (lane_iota + i * 16) < M
        plsc.addupdate_scatter(den_v, [relv], w, mask=vm)
        return 0
    lax.fori_loop(0, (M + 15) // 16, den_body, 0)

    nch = (M + KB - 1) // KB

    def _zcp(c, zb, sem):
        return pltpu.make_async_copy(z_hbm.at[srcL.at[pl.ds(c * KB, KB)]],
                                     zb, sem)

    @pl.when(nch > 0)
    def _():
        _zcp(0, zb0, sem_z0).start()

    @pl.when(nch > 1)
    def _():
        _zcp(1, zb1, sem_z1).start()

    def _process(c, zb):
        def alpha_g(g, _):
            i = c * (KB // 16) + g
            relv, w = _edge_w(i)
            den = plsc.load_gather(den_v, [relv]) + 1e-16
            a = w / den
            vm = (lane_iota + i * 16) < M
            a = jnp.where(vm, a, 0.0)
            abuf[pl.ds(g * 16, 16)] = a
            return 0
        lax.fori_loop(0, KB // 16, alpha_g, 0, unroll=4)

        @plsc.parallel_loop(0, KB // 16)
        def group_body(g):
            av = abuf[pl.ds(g * 16, 16)]
            relv = relL[pl.ds(c * KB + g * 16, 16)]
            for j in range(16):
                a = av[j]
                rel = relv[j]
                zvs = [zb[g * 16 + j, pl.ds(seg * 16, 16)]
                       for seg in range(D // 16)]
                for seg in range(D // 16):
                    plsc.addupdate(h_loc.at[rel, pl.ds(seg * 16, 16)],
                                   a * zvs[seg])

    def chunk_pair(c2, _):
        c = c2 * 2
        _zcp(c, zb0, sem_z0).wait()
        _process(c, zb0)

        @pl.when(c + 2 < nch)
        def _():
            _zcp(c + 2, zb0, sem_z0).start()

        @pl.when(c + 1 < nch)
        def _():
            _zcp(c + 1, zb1, sem_z1).wait()
            _process(c + 1, zb1)

            @pl.when(c + 3 < nch)
            def _():
                _zcp(c + 3, zb1, sem_z1).start()
        return 0

    lax.fori_loop(0, (nch + 1) // 2, chunk_pair, 0)

    pltpu.sync_copy(h_loc, out_hbm.at[pl.ds(base, NB)])


_sc_mesh = plsc.VectorSubcoreMesh(core_axis_name="c", subcore_axis_name="s")

_sc_params = pltpu.CompilerParams()
if "needs_layout_passes" in pltpu.CompilerParams.__dataclass_fields__:
    _sc_params = dataclasses.replace(_sc_params, needs_layout_passes=False)

_gat_sc = functools.partial(
    pl.kernel,
    out_type=jax.ShapeDtypeStruct((NP, D), jnp.float32),
    mesh=_sc_mesh,
    compiler_params=_sc_params,
    scratch_types=[
        pltpu.VMEM((NP,), jnp.float32),
        pltpu.VMEM((NP,), jnp.float32),
        pltpu.VMEM((NB,), jnp.float32),
        pltpu.VMEM((CAPL,), jnp.int32),
        pltpu.VMEM((CAPL,), jnp.int32),
        pltpu.VMEM((NB, D), jnp.float32),
        pltpu.VMEM((KB, D), jnp.float32),
        pltpu.VMEM((KB, D), jnp.float32),
        pltpu.VMEM((SC_CH,), jnp.int32),
        pltpu.VMEM((SC_CH,), jnp.int32),
        pltpu.VMEM((SC_CH,), jnp.int32),
        pltpu.VMEM((SC_CH,), jnp.int32),
        pltpu.VMEM((KB,), jnp.float32),
        pltpu.SemaphoreType.DMA,
        pltpu.SemaphoreType.DMA,
        pltpu.SemaphoreType.DMA,
        pltpu.SemaphoreType.DMA,
        pltpu.SemaphoreType.DMA,
        pltpu.SemaphoreType.DMA,
    ],
)(_gat_sc_body)


def kernel(x, edge_index, W_fc, W_attn):
    xp = jnp.pad(x, ((0, NP - N), (0, 0)))
    z, as2, ad2 = _tc_prep(xp, W_fc, W_attn)
    h = _gat_sc(z, as2, ad2, edge_index[0], edge_index[1])
    return h[:N]
